# baseline (speedup 1.0000x reference)
import functools

import jax
import jax.numpy as jnp
from jax import lax
from jax.experimental import pallas as pl
from jax.experimental.pallas import tpu as pltpu
from jax.experimental.pallas import tpu_sc as plsc

BATCH = 1024
VOCAB = 100000
NW = 32
ROWS_PER_TILE = BATCH // NW
CHUNK = VOCAB // 2
LANES = 16
VREGS_H = CHUNK // LANES
UNROLL = 25
BIG = 2 ** 30

_mesh = plsc.VectorSubcoreMesh(core_axis_name="c", subcore_axis_name="s")


@functools.partial(
    pl.kernel,
    mesh=_mesh,
    compiler_params=pltpu.CompilerParams(needs_layout_passes=False),
    out_type=(
        jax.ShapeDtypeStruct((BATCH,), jnp.float32),
        jax.ShapeDtypeStruct((BATCH,), jnp.float32),
    ),
    scratch_types=[
        pltpu.VMEM((CHUNK,), jnp.float32),
        pltpu.VMEM((CHUNK,), jnp.float32),
        pltpu.VMEM((ROWS_PER_TILE,), jnp.int32),
        pltpu.VMEM((ROWS_PER_TILE,), jnp.float32),
        pltpu.VMEM((ROWS_PER_TILE,), jnp.float32),
        pltpu.SemaphoreType.DMA,
        pltpu.SemaphoreType.DMA,
        pltpu.SemaphoreType.DMA,
    ],
)
def _vocab_pp(inp_hbm, tab_hbm, cast_hbm, maxp_hbm,
              buf0, buf1, idx_v, maxp_v, cast_v, sem0, sem1, gsem):
    wid = lax.axis_index("s") * 2 + lax.axis_index("c")
    row0 = wid * ROWS_PER_TILE
    bufs = (buf0, buf1)
    sems = (sem0, sem1)
    lane = lax.broadcasted_iota(jnp.int32, (LANES,), 0)

    def dma(r, h):
        return pltpu.make_async_copy(
            inp_hbm.at[pl.ds((row0 + r) * VOCAB + h * CHUNK, CHUNK)],
            bufs[h], sems[h])

    dma(0, 0).start()
    dma(0, 1).start()

    def scan_half(bref, h, acc):
        base = h * CHUNK

        def body(j, acc):
            mv0, bi0, mv1, bi1 = acc
            off = j * (UNROLL * LANES)
            idxb = lane + (base + off)
            for u in range(UNROLL):
                v = bref[pl.ds(off + u * LANES, LANES)]
                iv = idxb + (u * LANES)
                if u % 2 == 0:
                    m = v > mv0
                    mv0 = jnp.where(m, v, mv0)
                    bi0 = jnp.where(m, iv, bi0)
                else:
                    m = v > mv1
                    mv1 = jnp.where(m, v, mv1)
                    bi1 = jnp.where(m, iv, bi1)
            return (mv0, bi0, mv1, bi1)

        return lax.fori_loop(0, VREGS_H // UNROLL, body, acc)

    def row_body(r, carry):
        ninf = jnp.full((LANES,), -jnp.inf, jnp.float32)
        zz = jnp.zeros((LANES,), jnp.int32)
        acc = (ninf, zz, ninf, zz)
        rn = jnp.minimum(r + 1, ROWS_PER_TILE - 1)
        dma(r, 0).wait()
        acc = scan_half(buf0, 0, acc)
        dma(rn, 0).start()
        dma(r, 1).wait()
        acc = scan_half(buf1, 1, acc)
        dma(rn, 1).start()
        mv0, bi0, mv1, bi1 = acc
        m = jnp.maximum(jnp.max(mv0), jnp.max(mv1))
        cand = jnp.minimum(jnp.where(mv0 == m, bi0, BIG),
                           jnp.where(mv1 == m, bi1, BIG))
        amin = jnp.min(cand)
        rvec = jnp.full((LANES,), r, jnp.int32)
        msk0 = lane == 0
        plsc.store_scatter(maxp_v, [rvec], jnp.full((LANES,), m, jnp.float32),
                           mask=msk0)
        plsc.store_scatter(idx_v, [rvec], jnp.full((LANES,), amin, jnp.int32),
                           mask=msk0)
        return carry

    lax.fori_loop(0, ROWS_PER_TILE, row_body, 0)
    dma(ROWS_PER_TILE - 1, 0).wait()
    dma(ROWS_PER_TILE - 1, 1).wait()

    g = pltpu.make_async_copy(tab_hbm.at[idx_v], cast_v, gsem)
    g.start()
    g.wait()
    pltpu.sync_copy(cast_v, cast_hbm.at[pl.ds(row0, ROWS_PER_TILE)])
    pltpu.sync_copy(maxp_v, maxp_hbm.at[pl.ds(row0, ROWS_PER_TILE)])


def kernel(input, table_values):
    return _vocab_pp(input.reshape(-1), table_values)
